# baseline (speedup 1.0000x reference)
import jax
import jax.numpy as jnp
from jax import lax
from jax.experimental import pallas as pl
from jax.experimental.pallas import tpu as pltpu
from jax.experimental.pallas import tpu_sc as plsc

_M = 6
_B = 1024
_H = 128
_NC = 2
_NS = 16
_NW = _NC * _NS
_SPW = _B // _NW
_C = 256
_OFFLEN = 1040


def _pool_body(x_hbm, split_hbm, off_hbm, h_hbm, r_hbm, off_v, h_v, r_v,
               st_v, xa, sp):
    n_total = x_hbm.shape[0]
    wid = lax.axis_index("s") * _NC + lax.axis_index("c")
    s0 = wid * _SPW
    pltpu.sync_copy(off_hbm.at[pl.ds(s0, 48)], off_v)
    pltpu.sync_copy(h_hbm.at[pl.ds(s0, _SPW)], h_v)

    neg_inf = jnp.full((16,), -jnp.inf, jnp.float32)
    zero16 = jnp.zeros((16,), jnp.float32)
    iota16 = lax.iota(jnp.int32, 16)
    _dnums = lax.GatherDimensionNumbers(
        offset_dims=(), collapsed_slice_dims=(0,), start_index_map=(0,))

    def hsum16(p):
        for k in (1, 2, 4, 8):
            perm = lax.gather(
                p, (iota16 ^ k)[:, None], dimension_numbers=_dnums,
                slice_sizes=(1,),
                mode=lax.GatherScatterMode.PROMISE_IN_BOUNDS)
            p = p + perm
        return p

    def zero_body(s, _):
        for k in range(8):
            r_v[s, pl.ds(k * 16, 16)] = zero16
        st_v[s, pl.ds(0, 16)] = zero16
        st_v[s, pl.ds(16, 16)] = neg_inf
        return 0

    lax.fori_loop(0, _SPW, zero_body, 0, unroll=False)

    ov = off_v[pl.ds(0, 16)]
    a0 = ov[0]
    a1 = off_v[pl.ds(_SPW, 16)][0]
    nchunks = lax.div(a1 - a0 + (_C - 1), _C)

    def phase1(row):
        sval = sp[pl.ds(row, 16)][0] - s0
        sval = jnp.minimum(jnp.maximum(sval, 0), _SPW - 1)
        hx = tuple(h_v[sval, pl.ds(k * 16, 16)] for k in range(8))
        xr = tuple(xa[row, pl.ds(k * 16, 16)] for k in range(8))
        t = [xr[k] * hx[k] for k in range(8)]
        t = [t[0] + t[1], t[2] + t[3], t[4] + t[5], t[6] + t[7]]
        p = (t[0] + t[1]) + (t[2] + t[3])
        return sval, hsum16(p), xr

    def phase2(data, prev, m, d, r):
        sval, e_v, xr = data
        changed = sval != prev
        m = jnp.where(changed, neg_inf, m)
        d = jnp.where(changed, zero16, d)
        r = tuple(jnp.where(changed, zero16, r[k]) for k in range(8))
        m_new = jnp.maximum(m, e_v)
        scale = jnp.exp(m - m_new)
        w = jnp.exp(e_v - m_new)
        d = d * scale + w
        r = tuple(r[k] * scale + w * xr[k] for k in range(8))
        for k in range(8):
            r_v[sval, pl.ds(k * 16, 16)] = r[k]
        st_v[sval, pl.ds(0, 16)] = d
        st_v[sval, pl.ds(16, 16)] = m_new
        return (sval, m_new, d, r)

    _G = 4

    def chunk_body(ci, st):
        prev, m, d, r = st
        base = a0 + ci * _C
        abase = jnp.minimum((base // 8) * 8, n_total - _C - 8)
        shift = base - abase
        pltpu.sync_copy(x_hbm.at[pl.ds(abase, _C + 8)],
                        xa.at[pl.ds(0, _C + 8)])
        pltpu.sync_copy(split_hbm.at[pl.ds(abase, _C + 8)],
                        sp.at[pl.ds(0, _C + 8)])
        cnt = jnp.minimum(_C, a1 - base)
        ngrp = cnt // _G

        datas0 = tuple(phase1(j + shift) for j in range(_G))

        def group_body(g, st2):
            prev2, m2, d2, r2, datas = st2
            row0 = (g + 1) * _G + shift
            nxt = tuple(phase1(row0 + j) for j in range(_G))
            for j in range(_G):
                prev2, m2, d2, r2 = phase2(datas[j], prev2, m2, d2, r2)
            return (prev2, m2, d2, r2, nxt)

        prev, m, d, r, _ = lax.fori_loop(
            0, ngrp, group_body, (prev, m, d, r, datas0), unroll=False)
        st2 = (prev, m, d, r)

        def tail_body(i, st3):
            prev3, m3, d3, r3 = st3
            row = ngrp * _G + i + shift
            return phase2(phase1(row), prev3, m3, d3, r3)

        prev, m, d, r = lax.fori_loop(0, cnt - ngrp * _G, tail_body, st2,
                                      unroll=False)
        return (prev, m, d, r)

    st0 = (jnp.int32(-1), neg_inf, zero16, (zero16,) * 8)
    lax.fori_loop(0, nchunks, chunk_body, st0, unroll=False)

    def norm_body(s, _):
        d = st_v[s, pl.ds(0, 16)]
        mm = st_v[s, pl.ds(16, 16)]
        dfin = d + jnp.exp(-1000.0 - mm)
        for k in range(8):
            rk = r_v[s, pl.ds(k * 16, 16)]
            r_v[s, pl.ds(k * 16, 16)] = jnp.where(d > 0.0, rk / dfin, 0.0)
        return 0

    lax.fori_loop(0, _SPW, norm_body, 0, unroll=False)
    pltpu.sync_copy(r_v, r_hbm.at[pl.ds(s0, _SPW)])


_sc_mesh = plsc.VectorSubcoreMesh(
    core_axis_name="c", subcore_axis_name="s",
    num_cores=_NC, num_subcores=_NS)

_pool = pl.kernel(
    _pool_body,
    out_type=jax.ShapeDtypeStruct((_B, _H), jnp.float32),
    mesh=_sc_mesh,
    scratch_types=[
        pltpu.VMEM((48,), jnp.int32),
        pltpu.VMEM((_SPW, _H), jnp.float32),
        pltpu.VMEM((_SPW, _H), jnp.float32),
        pltpu.VMEM((_SPW, _H), jnp.float32),
        pltpu.VMEM((_C + 16, _H), jnp.float32),
        pltpu.VMEM((_C + 40,), jnp.int32),
    ],
)


def _lstm_body(h_ref, r_ref, c_ref, U_ref, b_ref, ho_ref, co_ref):
    h = h_ref[...]
    r = r_ref[...]
    c = c_ref[...]
    U = U_ref[...]
    b = b_ref[...]
    z = (jnp.dot(h, U[:_H, :], preferred_element_type=jnp.float32)
         + jnp.dot(r, U[_H:, :], preferred_element_type=jnp.float32) + b)
    i = jax.nn.sigmoid(z[:, :_H])
    f = jax.nn.sigmoid(z[:, _H:2 * _H])
    o = jax.nn.sigmoid(z[:, 2 * _H:3 * _H])
    g = jnp.tanh(z[:, 3 * _H:])
    c_new = f * c + i * g
    co_ref[...] = c_new
    ho_ref[...] = o * jnp.tanh(c_new)


_lstm = pl.pallas_call(
    _lstm_body,
    out_shape=(jax.ShapeDtypeStruct((_B, _H), jnp.float32),
               jax.ShapeDtypeStruct((_B, _H), jnp.float32)),
)


def kernel(atom_features, atom_split, U, b):
    x = atom_features
    split = atom_split.astype(jnp.int32)
    n = x.shape[0]
    tvals = jnp.arange(_B + 1, dtype=jnp.int32)
    off = jnp.sum((split[None, :] < tvals[:, None]).astype(jnp.int32), axis=1)
    off_pad = jnp.concatenate(
        [off, jnp.full((_OFFLEN - (_B + 1),), n, jnp.int32)])
    b2 = b.reshape(1, 4 * _H)
    h = jnp.zeros((_B, _H), jnp.float32)
    c = jnp.zeros((_B, _H), jnp.float32)
    r = None
    for step in range(_M):
        r = _pool(x, split, off_pad, h)
        if step < _M - 1:
            h, c = _lstm(h, r, c, U, b2)
    return jnp.concatenate([h, r], axis=1)


# speedup vs baseline: 1.4591x; 1.4591x over previous
import jax
import jax.numpy as jnp
from jax import lax
from jax.experimental import pallas as pl
from jax.experimental.pallas import tpu as pltpu
from jax.experimental.pallas import tpu_sc as plsc

_M = 6
_B = 1024
_H = 128
_NC = 2
_NS = 16
_NW = _NC * _NS
_SPW = _B // _NW
_C = 256
_OFFLEN = 1040


def _pool_body(x_hbm, split_hbm, off_hbm, h_hbm, r_hbm, off_v, h_v, r_v,
               st_v, xa, sp):
    n_total = x_hbm.shape[0]
    wid = lax.axis_index("s") * _NC + lax.axis_index("c")
    s0 = wid * _SPW
    pltpu.sync_copy(off_hbm.at[pl.ds(s0, 48)], off_v)
    pltpu.sync_copy(h_hbm.at[pl.ds(s0, _SPW)], h_v)

    neg_inf = jnp.full((16,), -jnp.inf, jnp.float32)
    zero16 = jnp.zeros((16,), jnp.float32)
    iota16 = lax.iota(jnp.int32, 16)
    _dnums = lax.GatherDimensionNumbers(
        offset_dims=(), collapsed_slice_dims=(0,), start_index_map=(0,))

    def hsum16(p):
        for k in (1, 2, 4, 8):
            perm = lax.gather(
                p, (iota16 ^ k)[:, None], dimension_numbers=_dnums,
                slice_sizes=(1,),
                mode=lax.GatherScatterMode.PROMISE_IN_BOUNDS)
            p = p + perm
        return p

    def zero_body(s, _):
        for k in range(8):
            r_v[s, pl.ds(k * 16, 16)] = zero16
        st_v[s, pl.ds(0, 16)] = zero16
        st_v[s, pl.ds(16, 16)] = neg_inf
        return 0

    lax.fori_loop(0, _SPW, zero_body, 0, unroll=False)

    ov = off_v[pl.ds(0, 16)]
    a0 = ov[0]
    a1 = off_v[pl.ds(_SPW, 16)][0]
    nchunks = lax.div(a1 - a0 + (_C - 1), _C)

    def phase1(row):
        sval = sp[pl.ds(row, 16)][0] - s0
        sval = jnp.minimum(jnp.maximum(sval, 0), _SPW - 1)
        hx = tuple(h_v[sval, pl.ds(k * 16, 16)] for k in range(8))
        xr = tuple(xa[row, pl.ds(k * 16, 16)] for k in range(8))
        t = [xr[k] * hx[k] for k in range(8)]
        t = [t[0] + t[1], t[2] + t[3], t[4] + t[5], t[6] + t[7]]
        p = (t[0] + t[1]) + (t[2] + t[3])
        return sval, hsum16(p)

    def phase2(data, row, prev, m, d, r):
        sval, e_v = data
        xr = tuple(xa[row, pl.ds(k * 16, 16)] for k in range(8))
        changed = sval != prev
        m = jnp.where(changed, neg_inf, m)
        d = jnp.where(changed, zero16, d)
        r = tuple(jnp.where(changed, zero16, r[k]) for k in range(8))
        m_new = jnp.maximum(m, e_v)
        scale = jnp.exp(m - m_new)
        w = jnp.exp(e_v - m_new)
        d = d * scale + w
        r = tuple(r[k] * scale + w * xr[k] for k in range(8))
        for k in range(8):
            r_v[sval, pl.ds(k * 16, 16)] = r[k]
        st_v[sval, pl.ds(0, 16)] = d
        st_v[sval, pl.ds(16, 16)] = m_new
        return (sval, m_new, d, r)

    _G = 4

    def chunk_body(ci, st):
        prev, m, d, r = st
        base = a0 + ci * _C
        abase = jnp.minimum((base // 8) * 8, n_total - _C - 8)
        shift = base - abase
        pltpu.sync_copy(x_hbm.at[pl.ds(abase, _C + 8)],
                        xa.at[pl.ds(0, _C + 8)])
        pltpu.sync_copy(split_hbm.at[pl.ds(abase, _C + 8)],
                        sp.at[pl.ds(0, _C + 8)])
        cnt = jnp.minimum(_C, a1 - base)
        ngrp = cnt // _G

        datas0 = tuple(phase1(j + shift) for j in range(_G))

        def group_body(g, st2):
            prev2, m2, d2, r2, datas = st2
            row0 = (g + 1) * _G + shift
            nxt = tuple(phase1(row0 + j) for j in range(_G))
            rowp = g * _G + shift
            for j in range(_G):
                prev2, m2, d2, r2 = phase2(datas[j], rowp + j,
                                           prev2, m2, d2, r2)
            return (prev2, m2, d2, r2, nxt)

        prev, m, d, r, _ = lax.fori_loop(
            0, ngrp, group_body, (prev, m, d, r, datas0), unroll=False)
        st2 = (prev, m, d, r)

        def tail_body(i, st3):
            prev3, m3, d3, r3 = st3
            row = ngrp * _G + i + shift
            return phase2(phase1(row), row, prev3, m3, d3, r3)

        prev, m, d, r = lax.fori_loop(0, cnt - ngrp * _G, tail_body, st2,
                                      unroll=False)
        return (prev, m, d, r)

    st0 = (jnp.int32(-1), neg_inf, zero16, (zero16,) * 8)
    lax.fori_loop(0, nchunks, chunk_body, st0, unroll=False)

    def norm_body(s, _):
        d = st_v[s, pl.ds(0, 16)]
        mm = st_v[s, pl.ds(16, 16)]
        dfin = d + jnp.exp(-1000.0 - mm)
        for k in range(8):
            rk = r_v[s, pl.ds(k * 16, 16)]
            r_v[s, pl.ds(k * 16, 16)] = jnp.where(d > 0.0, rk / dfin, 0.0)
        return 0

    lax.fori_loop(0, _SPW, norm_body, 0, unroll=False)
    pltpu.sync_copy(r_v, r_hbm.at[pl.ds(s0, _SPW)])


_sc_mesh = plsc.VectorSubcoreMesh(
    core_axis_name="c", subcore_axis_name="s",
    num_cores=_NC, num_subcores=_NS)

_pool = pl.kernel(
    _pool_body,
    out_type=jax.ShapeDtypeStruct((_B, _H), jnp.float32),
    mesh=_sc_mesh,
    scratch_types=[
        pltpu.VMEM((48,), jnp.int32),
        pltpu.VMEM((_SPW, _H), jnp.float32),
        pltpu.VMEM((_SPW, _H), jnp.float32),
        pltpu.VMEM((_SPW, _H), jnp.float32),
        pltpu.VMEM((_C + 16, _H), jnp.float32),
        pltpu.VMEM((_C + 40,), jnp.int32),
    ],
)


def _lstm_body(h_ref, r_ref, c_ref, U_ref, b_ref, ho_ref, co_ref):
    h = h_ref[...]
    r = r_ref[...]
    c = c_ref[...]
    U = U_ref[...]
    b = b_ref[...]
    z = (jnp.dot(h, U[:_H, :], preferred_element_type=jnp.float32)
         + jnp.dot(r, U[_H:, :], preferred_element_type=jnp.float32) + b)
    i = jax.nn.sigmoid(z[:, :_H])
    f = jax.nn.sigmoid(z[:, _H:2 * _H])
    o = jax.nn.sigmoid(z[:, 2 * _H:3 * _H])
    g = jnp.tanh(z[:, 3 * _H:])
    c_new = f * c + i * g
    co_ref[...] = c_new
    ho_ref[...] = o * jnp.tanh(c_new)


_lstm = pl.pallas_call(
    _lstm_body,
    out_shape=(jax.ShapeDtypeStruct((_B, _H), jnp.float32),
               jax.ShapeDtypeStruct((_B, _H), jnp.float32)),
)


def kernel(atom_features, atom_split, U, b):
    x = atom_features
    split = atom_split.astype(jnp.int32)
    n = x.shape[0]
    tvals = jnp.arange(_B + 1, dtype=jnp.int32)
    off = jnp.sum((split[None, :] < tvals[:, None]).astype(jnp.int32), axis=1)
    off_pad = jnp.concatenate(
        [off, jnp.full((_OFFLEN - (_B + 1),), n, jnp.int32)])
    b2 = b.reshape(1, 4 * _H)
    h = jnp.zeros((_B, _H), jnp.float32)
    c = jnp.zeros((_B, _H), jnp.float32)
    r = None
    for step in range(_M):
        r = _pool(x, split, off_pad, h)
        if step < _M - 1:
            h, c = _lstm(h, r, c, U, b2)
    return jnp.concatenate([h, r], axis=1)


# speedup vs baseline: 1.8282x; 1.2530x over previous
import jax
import jax.numpy as jnp
from jax import lax
from jax.experimental import pallas as pl
from jax.experimental.pallas import tpu as pltpu
from jax.experimental.pallas import tpu_sc as plsc

_M = 6
_B = 1024
_H = 128
_NC = 2
_NS = 16
_NW = _NC * _NS
_SPW = _B // _NW
_C = 256
_OFFLEN = 1040


def _pool_body(x_hbm, split_hbm, off_hbm, h_hbm, r_hbm, off_v, h_v, r_v,
               st_v, xa0, xa1, sp0, sp1, semA, semB):
    n_total = x_hbm.shape[0]
    wid = lax.axis_index("s") * _NC + lax.axis_index("c")
    s0 = wid * _SPW
    pltpu.sync_copy(off_hbm.at[pl.ds(s0, 48)], off_v)
    pltpu.sync_copy(h_hbm.at[pl.ds(s0, _SPW)], h_v)

    neg_inf = jnp.full((16,), -jnp.inf, jnp.float32)
    zero16 = jnp.zeros((16,), jnp.float32)
    iota16 = lax.iota(jnp.int32, 16)
    _dnums = lax.GatherDimensionNumbers(
        offset_dims=(), collapsed_slice_dims=(0,), start_index_map=(0,))

    def hsum16(p):
        for k in (1, 2, 4, 8):
            perm = lax.gather(
                p, (iota16 ^ k)[:, None], dimension_numbers=_dnums,
                slice_sizes=(1,),
                mode=lax.GatherScatterMode.PROMISE_IN_BOUNDS)
            p = p + perm
        return p

    def zero_body(s, _):
        for k in range(8):
            r_v[s, pl.ds(k * 16, 16)] = zero16
        st_v[s, pl.ds(0, 16)] = zero16
        st_v[s, pl.ds(16, 16)] = neg_inf
        return 0

    lax.fori_loop(0, _SPW, zero_body, 0, unroll=False)

    ov = off_v[pl.ds(0, 16)]
    a0 = ov[0]
    a1 = off_v[pl.ds(_SPW, 16)][0]
    nchunks = lax.div(a1 - a0 + (_C - 1), _C)

    def phase1(xbuf, spbuf, row):
        sval = spbuf[pl.ds(row, 16)][0] - s0
        sval = jnp.minimum(jnp.maximum(sval, 0), _SPW - 1)
        hx = tuple(h_v[sval, pl.ds(k * 16, 16)] for k in range(8))
        xr = tuple(xbuf[row, pl.ds(k * 16, 16)] for k in range(8))
        t = [xr[k] * hx[k] for k in range(8)]
        t = [t[0] + t[1], t[2] + t[3], t[4] + t[5], t[6] + t[7]]
        p = (t[0] + t[1]) + (t[2] + t[3])
        return sval, hsum16(p)

    def phase2(xbuf, data, row, prev, m, d, r):
        sval, e_v = data
        xr = tuple(xbuf[row, pl.ds(k * 16, 16)] for k in range(8))
        changed = sval != prev
        m = jnp.where(changed, neg_inf, m)
        d = jnp.where(changed, zero16, d)
        r = tuple(jnp.where(changed, zero16, r[k]) for k in range(8))
        m_new = jnp.maximum(m, e_v)
        scale = jnp.exp(m - m_new)
        w = jnp.exp(e_v - m_new)
        d = d * scale + w
        r = tuple(r[k] * scale + w * xr[k] for k in range(8))
        for k in range(8):
            r_v[sval, pl.ds(k * 16, 16)] = r[k]
        st_v[sval, pl.ds(0, 16)] = d
        st_v[sval, pl.ds(16, 16)] = m_new
        return (sval, m_new, d, r)

    _G = 4

    def dma_descs(ci, xbuf, spbuf, sem):
        base = a0 + ci * _C
        abase = jnp.minimum((base // 8) * 8, n_total - _C - 8)
        cx = pltpu.make_async_copy(x_hbm.at[pl.ds(abase, _C + 8)],
                                   xbuf.at[pl.ds(0, _C + 8)], sem)
        cs = pltpu.make_async_copy(split_hbm.at[pl.ds(abase, _C + 8)],
                                   spbuf.at[pl.ds(0, _C + 8)], sem)
        return cx, cs

    def dma_start(ci, xbuf, spbuf, sem):
        cx, cs = dma_descs(ci, xbuf, spbuf, sem)
        cx.start()
        cs.start()

    def dma_wait(ci, xbuf, spbuf, sem):
        cx, cs = dma_descs(ci, xbuf, spbuf, sem)
        cx.wait()
        cs.wait()

    def process_chunk(ci, xbuf, spbuf, st):
        prev, m, d, r = st
        base = a0 + ci * _C
        abase = jnp.minimum((base // 8) * 8, n_total - _C - 8)
        shift = base - abase
        cnt = jnp.maximum(jnp.minimum(_C, a1 - base), 0)
        ngrp = cnt // _G

        datas0 = tuple(phase1(xbuf, spbuf, j + shift) for j in range(_G))

        def group_body(g, st2):
            prev2, m2, d2, r2, datas = st2
            row0 = (g + 1) * _G + shift
            nxt = tuple(phase1(xbuf, spbuf, row0 + j) for j in range(_G))
            rowp = g * _G + shift
            for j in range(_G):
                prev2, m2, d2, r2 = phase2(xbuf, datas[j], rowp + j,
                                           prev2, m2, d2, r2)
            return (prev2, m2, d2, r2, nxt)

        prev, m, d, r, _ = lax.fori_loop(
            0, ngrp, group_body, (prev, m, d, r, datas0), unroll=False)
        st2 = (prev, m, d, r)

        def tail_body(i, st3):
            prev3, m3, d3, r3 = st3
            row = ngrp * _G + i + shift
            return phase2(xbuf, phase1(xbuf, spbuf, row), row,
                          prev3, m3, d3, r3)

        return lax.fori_loop(0, cnt - ngrp * _G, tail_body, st2,
                             unroll=False)

    dma_start(0, xa0, sp0, semA)
    npairs = lax.div(nchunks + 1, 2)

    def pair_body(t, st):
        ci0 = 2 * t
        dma_start(ci0 + 1, xa1, sp1, semB)
        dma_wait(ci0, xa0, sp0, semA)
        st = process_chunk(ci0, xa0, sp0, st)
        dma_start(ci0 + 2, xa0, sp0, semA)
        dma_wait(ci0 + 1, xa1, sp1, semB)
        st = process_chunk(ci0 + 1, xa1, sp1, st)
        return st

    st0 = (jnp.int32(-1), neg_inf, zero16, (zero16,) * 8)
    lax.fori_loop(0, npairs, pair_body, st0, unroll=False)
    dma_wait(2 * npairs, xa0, sp0, semA)

    def norm_body(s, _):
        d = st_v[s, pl.ds(0, 16)]
        mm = st_v[s, pl.ds(16, 16)]
        dfin = d + jnp.exp(-1000.0 - mm)
        for k in range(8):
            rk = r_v[s, pl.ds(k * 16, 16)]
            r_v[s, pl.ds(k * 16, 16)] = jnp.where(d > 0.0, rk / dfin, 0.0)
        return 0

    lax.fori_loop(0, _SPW, norm_body, 0, unroll=False)
    pltpu.sync_copy(r_v, r_hbm.at[pl.ds(s0, _SPW)])


_sc_mesh = plsc.VectorSubcoreMesh(
    core_axis_name="c", subcore_axis_name="s",
    num_cores=_NC, num_subcores=_NS)

_pool = pl.kernel(
    _pool_body,
    out_type=jax.ShapeDtypeStruct((_B, _H), jnp.float32),
    mesh=_sc_mesh,
    scratch_types=[
        pltpu.VMEM((48,), jnp.int32),
        pltpu.VMEM((_SPW, _H), jnp.float32),
        pltpu.VMEM((_SPW, _H), jnp.float32),
        pltpu.VMEM((_SPW, _H), jnp.float32),
        pltpu.VMEM((_C + 16, _H), jnp.float32),
        pltpu.VMEM((_C + 16, _H), jnp.float32),
        pltpu.VMEM((_C + 40,), jnp.int32),
        pltpu.VMEM((_C + 40,), jnp.int32),
        pltpu.SemaphoreType.DMA,
        pltpu.SemaphoreType.DMA,
    ],
)


def _lstm_body(h_ref, r_ref, c_ref, U_ref, b_ref, ho_ref, co_ref):
    h = h_ref[...]
    r = r_ref[...]
    c = c_ref[...]
    U = U_ref[...]
    b = b_ref[...]
    z = (jnp.dot(h, U[:_H, :], preferred_element_type=jnp.float32)
         + jnp.dot(r, U[_H:, :], preferred_element_type=jnp.float32) + b)
    i = jax.nn.sigmoid(z[:, :_H])
    f = jax.nn.sigmoid(z[:, _H:2 * _H])
    o = jax.nn.sigmoid(z[:, 2 * _H:3 * _H])
    g = jnp.tanh(z[:, 3 * _H:])
    c_new = f * c + i * g
    co_ref[...] = c_new
    ho_ref[...] = o * jnp.tanh(c_new)


_lstm = pl.pallas_call(
    _lstm_body,
    out_shape=(jax.ShapeDtypeStruct((_B, _H), jnp.float32),
               jax.ShapeDtypeStruct((_B, _H), jnp.float32)),
)


def kernel(atom_features, atom_split, U, b):
    x = atom_features
    split = atom_split.astype(jnp.int32)
    n = x.shape[0]
    tvals = jnp.arange(_B + 1, dtype=jnp.int32)
    off = jnp.sum((split[None, :] < tvals[:, None]).astype(jnp.int32), axis=1)
    off_pad = jnp.concatenate(
        [off, jnp.full((_OFFLEN - (_B + 1),), n, jnp.int32)])
    b2 = b.reshape(1, 4 * _H)
    h = jnp.zeros((_B, _H), jnp.float32)
    c = jnp.zeros((_B, _H), jnp.float32)
    r = None
    for step in range(_M):
        r = _pool(x, split, off_pad, h)
        if step < _M - 1:
            h, c = _lstm(h, r, c, U, b2)
    return jnp.concatenate([h, r], axis=1)
